# baseline (speedup 1.0000x reference)
import functools

import jax
import jax.numpy as jnp
from jax import lax
from jax.experimental import pallas as pl
from jax.experimental.pallas import tpu as pltpu
from jax.experimental.pallas import tpu_sc as plsc

DIM = 1024
NUM_ROWS = 5
LANES = 16
NC, NS = 2, 16
NW = NC * NS
CH = 32
UNROLL = 8


def _sc_embed(ids_flat, tbl_flat, scale16, n):
    n_per_w = n // NW
    nch = n_per_w // CH
    mesh = plsc.VectorSubcoreMesh(
        core_axis_name="c", subcore_axis_name="s", num_cores=NC, num_subcores=NS
    )

    @functools.partial(
        pl.kernel,
        out_type=jax.ShapeDtypeStruct((n * DIM,), jnp.float32),
        mesh=mesh,
        compiler_params=pltpu.CompilerParams(needs_layout_passes=False),
        scratch_types=[
            pltpu.VMEM((n_per_w,), jnp.int32),
            pltpu.VMEM((LANES,), jnp.float32),
            pltpu.VMEM((NUM_ROWS * DIM,), jnp.float32),
            pltpu.VMEM((CH * DIM,), jnp.float32),
            pltpu.VMEM((CH * DIM,), jnp.float32),
            pltpu.SemaphoreType.DMA,
            pltpu.SemaphoreType.DMA,
        ],
    )
    def k(ids_hbm, tbl_hbm, scl_hbm, out_hbm, idx_v, scl_v, tbl_v,
          buf_a, buf_b, ssem_a, ssem_b):
        wid = lax.axis_index("s") * NC + lax.axis_index("c")
        base = wid * n_per_w
        pltpu.sync_copy(ids_hbm.at[pl.ds(base, n_per_w)], idx_v)
        pltpu.sync_copy(scl_hbm, scl_v)
        pltpu.sync_copy(tbl_hbm, tbl_v)
        sv = scl_v[...]

        def scale_slice(j, _):
            tbl_v[pl.ds(j * LANES, LANES)] = tbl_v[pl.ds(j * LANES, LANES)] * sv
            return 0
        lax.fori_loop(0, NUM_ROWS * DIM // LANES, scale_slice, 0)

        def shift_slice(j, _):
            idx_v[pl.ds(j * LANES, LANES)] = (
                idx_v[pl.ds(j * LANES, LANES)] << 10
            )
            return 0
        lax.fori_loop(0, n_per_w // LANES, shift_slice, 0)

        obase = lax.iota(jnp.int32, LANES) << 10

        def fill(buf, c):
            for rg in range(CH // LANES):
                a_in0 = idx_v[pl.ds((c * CH + rg * LANES), LANES)]
                a_out0 = obase + (rg * LANES * DIM)

                def col(j, carry, rg=rg):
                    a_in, a_out = carry
                    x = plsc.load_gather(tbl_v, [a_in])
                    plsc.store_scatter(buf, [a_out], x)
                    return a_in + 1, a_out + 1

                lax.fori_loop(0, DIM, col, (a_in0, a_out0), unroll=UNROLL)

        bufs = ((buf_a, ssem_a), (buf_b, ssem_b))

        def out_slice(c):
            return out_hbm.at[pl.ds((base + c * CH) * DIM, CH * DIM)]

        for b, (buf, ssem) in enumerate(bufs):
            fill(buf, b)
            pltpu.async_copy(buf, out_slice(b), ssem)

        def group(g, _):
            for b, (buf, ssem) in enumerate(bufs):
                c = 2 * g + b
                pltpu.make_async_copy(buf, out_slice(c - 2), ssem).wait()
                fill(buf, c)
                pltpu.async_copy(buf, out_slice(c), ssem)
            return 0

        lax.fori_loop(1, nch // 2, group, 0)

        for b, (buf, ssem) in enumerate(bufs):
            pltpu.make_async_copy(buf, out_slice(nch - 2 + b), ssem).wait()

    return k(ids_flat, tbl_flat, scale16)


def kernel(modality_ids, embed, scale):
    b, s = modality_ids.shape
    n = b * s
    ids_flat = modality_ids.reshape(n).astype(jnp.int32)
    tbl_flat = embed.astype(jnp.float32).reshape(NUM_ROWS * DIM)
    scale16 = jnp.broadcast_to(scale.astype(jnp.float32), (LANES,))
    out = _sc_embed(ids_flat, tbl_flat, scale16, n)
    return out.reshape(b, s, DIM)


# speedup vs baseline: 14.1843x; 14.1843x over previous
import functools

import jax
import jax.numpy as jnp
from jax import lax
from jax.experimental import pallas as pl
from jax.experimental.pallas import tpu as pltpu
from jax.experimental.pallas import tpu_sc as plsc

DIM = 1024
NUM_ROWS = 5
LANES = 16
NC, NS = 2, 16
NW = NC * NS
REP = 16
POSCAP = 1088
QMAX = 8
BIG = 1 << 30


def _sc_embed(ids_flat, tbl_flat, scale16, n):
    n_per_w = n // NW
    nvec = n_per_w // LANES
    mesh = plsc.VectorSubcoreMesh(
        core_axis_name="c", subcore_axis_name="s", num_cores=NC, num_subcores=NS
    )

    @functools.partial(
        pl.kernel,
        out_type=jax.ShapeDtypeStruct((n, DIM), jnp.float32),
        mesh=mesh,
        compiler_params=pltpu.CompilerParams(needs_layout_passes=False),
        scratch_types=[
            pltpu.VMEM((n_per_w,), jnp.int32),
            pltpu.VMEM((LANES,), jnp.float32),
            pltpu.VMEM((NUM_ROWS * DIM,), jnp.float32),
            pltpu.VMEM((NUM_ROWS * REP, DIM), jnp.float32),
            pltpu.VMEM((NUM_ROWS * POSCAP,), jnp.int32),
            pltpu.SemaphoreType.DMA,
        ],
    )
    def k(ids_hbm, tbl_hbm, scl_hbm, out_hbm, idx_v, scl_v, tbl_v,
          blk_v, pos_v, ssem):
        wid = lax.axis_index("s") * NC + lax.axis_index("c")
        base = wid * n_per_w
        pltpu.sync_copy(ids_hbm.at[pl.ds(base, n_per_w)], idx_v)
        pltpu.sync_copy(scl_hbm, scl_v)
        pltpu.sync_copy(tbl_hbm, tbl_v)
        sv = scl_v[...]
        iota16 = lax.iota(jnp.int32, LANES)

        def scale_slice(j, _):
            tbl_v[pl.ds(j * LANES, LANES)] = tbl_v[pl.ds(j * LANES, LANES)] * sv
            return 0
        lax.fori_loop(0, NUM_ROWS * DIM // LANES, scale_slice, 0)

        for m in range(NUM_ROWS):
            def rep_body(r, _, m=m):
                def cp_r(j, _, m=m):
                    blk_v[m * REP + r, pl.ds(j * LANES, LANES)] = tbl_v[
                        pl.ds(m * DIM + j * LANES, LANES)
                    ]
                    return 0
                lax.fori_loop(0, DIM // LANES, cp_r, 0, unroll=8)
                return 0
            lax.fori_loop(0, REP, rep_body, 0)

        counts = []
        for m in range(NUM_ROWS):
            start = m * POSCAP

            def comp(v, cnt, m=m, start=start):
                ids16 = idx_v[pl.ds(v * LANES, LANES)]
                mask = ids16 == m
                posv = (base + v * LANES) + iota16
                plsc.store_compressed(pos_v.at[pl.ds(start + cnt, LANES)],
                                      posv, mask=mask)
                return cnt + jnp.sum(mask.astype(jnp.int32))

            cnt = lax.fori_loop(0, nvec, comp, jnp.int32(0))

            fl = (cnt >> 4) << 4
            head = pos_v[pl.ds(start, LANES)]
            valid_head = jnp.where(iota16 < jnp.minimum(cnt, LANES), head, BIG)
            pad = jnp.broadcast_to(jnp.min(valid_head), (LANES,))
            tail = pos_v[pl.ds(start + fl, LANES)]
            pos_v[pl.ds(start + fl, LANES)] = jnp.where(
                iota16 < (cnt & 15), tail, pad
            )
            counts.append(cnt)

        def wait_one():
            pltpu.make_async_copy(
                blk_v.at[pl.ds(0, REP)], out_hbm.at[pl.ds(0, REP)], ssem
            ).wait()

        state = (jnp.int32(0), jnp.int32(0))
        for m in range(NUM_ROWS):
            start = m * POSCAP
            t_m = (counts[m] + 15) >> 4

            def scat(t, carry, m=m, start=start):
                issued, waited = carry
                idxvec = pos_v[pl.ds(start + t * LANES, LANES)]
                pltpu.async_copy(
                    blk_v.at[pl.ds(m * REP, REP)], out_hbm.at[idxvec], ssem
                )
                issued = issued + 1

                def throttle(w):
                    wait_one()
                    return w + 1

                waited = lax.cond(issued - waited > QMAX, throttle,
                                  lambda w: w, waited)
                return issued, waited

            state = lax.fori_loop(0, t_m, scat, state)

        issued, waited = state

        def drain(i, _):
            wait_one()
            return 0
        lax.fori_loop(0, issued - waited, drain, 0)

    return k(ids_flat, tbl_flat, scale16)


def kernel(modality_ids, embed, scale):
    b, s = modality_ids.shape
    n = b * s
    ids_flat = modality_ids.reshape(n).astype(jnp.int32)
    tbl_flat = embed.astype(jnp.float32).reshape(NUM_ROWS * DIM)
    scale16 = jnp.broadcast_to(scale.astype(jnp.float32), (LANES,))
    out = _sc_embed(ids_flat, tbl_flat, scale16, n)
    return out.reshape(b, s, DIM)


# speedup vs baseline: 16.9217x; 1.1930x over previous
import functools

import jax
import jax.numpy as jnp
from jax import lax
from jax.experimental import pallas as pl
from jax.experimental.pallas import tpu as pltpu
from jax.experimental.pallas import tpu_sc as plsc

DIM = 1024
NUM_ROWS = 5
LANES = 16
NC, NS = 2, 16
NW = NC * NS
REP = 16
POSCAP = 1088
QMAX = 12
BIG = 1 << 30
BIGF = 3.4e38


def _sc_embed(ids_flat, tbl_flat, scale, n):
    n_per_w = n // NW
    nvec = n_per_w // LANES
    mesh = plsc.VectorSubcoreMesh(
        core_axis_name="c", subcore_axis_name="s", num_cores=NC, num_subcores=NS
    )

    @functools.partial(
        pl.kernel,
        out_type=jax.ShapeDtypeStruct((n, DIM), jnp.float32),
        mesh=mesh,
        compiler_params=pltpu.CompilerParams(needs_layout_passes=False),
        scratch_types=[
            pltpu.VMEM((n_per_w,), jnp.int32),
            pltpu.VMEM((LANES,), jnp.float32),
            pltpu.VMEM((NUM_ROWS * DIM,), jnp.float32),
            pltpu.VMEM((NUM_ROWS * REP, DIM), jnp.float32),
            pltpu.VMEM((NUM_ROWS * POSCAP,), jnp.int32),
            pltpu.SemaphoreType.DMA,
        ],
    )
    def k(ids_hbm, tbl_hbm, scl_hbm, out_hbm, idx_v, scl_v, tbl_v,
          blk_v, pos_v, ssem):
        wid = lax.axis_index("s") * NC + lax.axis_index("c")
        base = wid * n_per_w
        pltpu.sync_copy(ids_hbm.at[pl.ds(base, n_per_w)], idx_v)
        pltpu.sync_copy(scl_hbm, scl_v.at[pl.ds(0, 1)])
        pltpu.sync_copy(tbl_hbm, tbl_v)
        iota16 = lax.iota(jnp.int32, LANES)

        raw = scl_v[...]
        sval = jnp.min(jnp.where(iota16 == 0, raw, jnp.float32(BIGF)))
        sv = jnp.broadcast_to(sval, (LANES,))

        def scale_slice(j, _):
            tbl_v[pl.ds(j * LANES, LANES)] = tbl_v[pl.ds(j * LANES, LANES)] * sv
            return 0
        lax.fori_loop(0, NUM_ROWS * DIM // LANES, scale_slice, 0)

        def wait_one():
            pltpu.make_async_copy(
                blk_v.at[pl.ds(0, REP)], out_hbm.at[pl.ds(0, REP)], ssem
            ).wait()

        state = (jnp.int32(0), jnp.int32(0))
        for m in range(NUM_ROWS):
            start = m * POSCAP

            def rep_body(r, _, m=m):
                def cp_r(j, _, m=m):
                    blk_v[m * REP + r, pl.ds(j * LANES, LANES)] = tbl_v[
                        pl.ds(m * DIM + j * LANES, LANES)
                    ]
                    return 0
                lax.fori_loop(0, DIM // LANES, cp_r, 0, unroll=8)
                return 0
            lax.fori_loop(0, REP, rep_body, 0)

            def comp(v, cnt, m=m, start=start):
                ids16 = idx_v[pl.ds(v * LANES, LANES)]
                mask = ids16 == m
                posv = (base + v * LANES) + iota16
                plsc.store_compressed(pos_v.at[pl.ds(start + cnt, LANES)],
                                      posv, mask=mask)
                return cnt + jnp.sum(mask.astype(jnp.int32))

            cnt = lax.fori_loop(0, nvec, comp, jnp.int32(0))

            fl = (cnt >> 4) << 4
            head = pos_v[pl.ds(start, LANES)]
            valid_head = jnp.where(iota16 < jnp.minimum(cnt, LANES), head, BIG)
            pad = jnp.broadcast_to(jnp.min(valid_head), (LANES,))
            tail = pos_v[pl.ds(start + fl, LANES)]
            pos_v[pl.ds(start + fl, LANES)] = jnp.where(
                iota16 < (cnt & 15), tail, pad
            )

            t_m = (cnt + 15) >> 4

            def scat(t, carry, m=m, start=start):
                issued, waited = carry
                idxvec = pos_v[pl.ds(start + t * LANES, LANES)]
                pltpu.async_copy(
                    blk_v.at[pl.ds(m * REP, REP)], out_hbm.at[idxvec], ssem
                )
                issued = issued + 1

                def throttle(w):
                    wait_one()
                    return w + 1

                waited = lax.cond(issued - waited > QMAX, throttle,
                                  lambda w: w, waited)
                return issued, waited

            state = lax.fori_loop(0, t_m, scat, state)

        issued, waited = state

        def drain(i, _):
            wait_one()
            return 0
        lax.fori_loop(0, issued - waited, drain, 0)

    return k(ids_flat, tbl_flat, scale)


def kernel(modality_ids, embed, scale):
    b, s = modality_ids.shape
    n = b * s
    ids_flat = modality_ids.reshape(n).astype(jnp.int32)
    tbl_flat = embed.astype(jnp.float32).reshape(NUM_ROWS * DIM)
    out = _sc_embed(ids_flat, tbl_flat, scale.astype(jnp.float32), n)
    return out.reshape(b, s, DIM)


# speedup vs baseline: 17.9359x; 1.0599x over previous
import functools

import jax
import jax.numpy as jnp
from jax import lax
from jax.experimental import pallas as pl
from jax.experimental.pallas import tpu as pltpu
from jax.experimental.pallas import tpu_sc as plsc

DIM = 1024
NUM_ROWS = 5
LANES = 16
NC, NS = 2, 16
NW = NC * NS
REP = 16
POSCAP = 1088
QMAX = 24
BIG = 1 << 30
BIGF = 3.4e38


def _sc_embed(ids_flat, tbl_flat, scale, n):
    n_per_w = n // NW
    nvec = n_per_w // LANES
    mesh = plsc.VectorSubcoreMesh(
        core_axis_name="c", subcore_axis_name="s", num_cores=NC, num_subcores=NS
    )

    @functools.partial(
        pl.kernel,
        out_type=jax.ShapeDtypeStruct((n, DIM), jnp.float32),
        mesh=mesh,
        compiler_params=pltpu.CompilerParams(needs_layout_passes=False),
        scratch_types=[
            pltpu.VMEM((n_per_w,), jnp.int32),
            pltpu.VMEM((LANES,), jnp.float32),
            pltpu.VMEM((NUM_ROWS * DIM,), jnp.float32),
            pltpu.VMEM((NUM_ROWS * REP, DIM), jnp.float32),
            pltpu.VMEM((NUM_ROWS * POSCAP,), jnp.int32),
            pltpu.SemaphoreType.DMA,
        ],
    )
    def k(ids_hbm, tbl_hbm, scl_hbm, out_hbm, idx_v, scl_v, tbl_v,
          blk_v, pos_v, ssem):
        wid = lax.axis_index("s") * NC + lax.axis_index("c")
        base = wid * n_per_w
        pltpu.sync_copy(ids_hbm.at[pl.ds(base, n_per_w)], idx_v)
        pltpu.sync_copy(scl_hbm, scl_v.at[pl.ds(0, 1)])
        pltpu.sync_copy(tbl_hbm, tbl_v)
        iota16 = lax.iota(jnp.int32, LANES)

        raw = scl_v[...]
        sval = jnp.min(jnp.where(iota16 == 0, raw, jnp.float32(BIGF)))
        sv = jnp.broadcast_to(sval, (LANES,))

        def scale_slice(j, _):
            tbl_v[pl.ds(j * LANES, LANES)] = tbl_v[pl.ds(j * LANES, LANES)] * sv
            return 0
        lax.fori_loop(0, NUM_ROWS * DIM // LANES, scale_slice, 0)

        def wait_one():
            pltpu.make_async_copy(
                blk_v.at[pl.ds(0, REP)], out_hbm.at[pl.ds(0, REP)], ssem
            ).wait()

        def modality(m, state):
            start = m * POSCAP

            def rep_body(r, _):
                def cp_r(j, _):
                    blk_v[m * REP + r, pl.ds(j * LANES, LANES)] = tbl_v[
                        pl.ds(m * DIM + j * LANES, LANES)
                    ]
                    return 0
                lax.fori_loop(0, DIM // LANES, cp_r, 0, unroll=8)
                return 0
            lax.fori_loop(0, REP, rep_body, 0)

            def comp(v, cnt):
                ids16 = idx_v[pl.ds(v * LANES, LANES)]
                mask = ids16 == m
                posv = (base + v * LANES) + iota16
                plsc.store_compressed(pos_v.at[pl.ds(start + cnt, LANES)],
                                      posv, mask=mask)
                return cnt + jnp.sum(mask.astype(jnp.int32))

            cnt = lax.fori_loop(0, nvec, comp, jnp.int32(0))

            fl = (cnt >> 4) << 4
            head = pos_v[pl.ds(start, LANES)]
            valid_head = jnp.where(iota16 < jnp.minimum(cnt, LANES), head, BIG)
            pad = jnp.broadcast_to(jnp.min(valid_head), (LANES,))
            tail = pos_v[pl.ds(start + fl, LANES)]
            pos_v[pl.ds(start + fl, LANES)] = jnp.where(
                iota16 < (cnt & 15), tail, pad
            )

            t_m = (cnt + 15) >> 4

            def scat(t, carry):
                issued, waited = carry
                idxvec = pos_v[pl.ds(start + t * LANES, LANES)]
                pltpu.async_copy(
                    blk_v.at[pl.ds(m * REP, REP)], out_hbm.at[idxvec], ssem
                )
                issued = issued + 1

                def throttle(w):
                    wait_one()
                    return w + 1

                waited = lax.cond(issued - waited > QMAX, throttle,
                                  lambda w: w, waited)
                return issued, waited

            return lax.fori_loop(0, t_m, scat, state)

        state = lax.fori_loop(
            0, NUM_ROWS, modality, (jnp.int32(0), jnp.int32(0))
        )
        issued, waited = state

        def drain(i, _):
            wait_one()
            return 0
        lax.fori_loop(0, issued - waited, drain, 0)

    return k(ids_flat, tbl_flat, scale)


def kernel(modality_ids, embed, scale):
    b, s = modality_ids.shape
    n = b * s
    ids_flat = modality_ids.reshape(n).astype(jnp.int32)
    tbl_flat = embed.astype(jnp.float32).reshape(NUM_ROWS * DIM)
    out = _sc_embed(ids_flat, tbl_flat, scale.astype(jnp.float32), n)
    return out.reshape(b, s, DIM)
